# baseline (speedup 1.0000x reference)
import functools

import jax
import jax.numpy as jnp
from jax import lax
from jax.experimental import pallas as pl
from jax.experimental.pallas import tpu as pltpu
from jax.experimental.pallas import tpu_sc as plsc

_B = 4096
_C = 1000
_CPAD = 1024
_BLK = 128
_NC = 2
_NS = 16
_NW = _NC * _NS
_LANES = 16
_SLABS_PER_W = _CPAD // (_NW * _LANES)
_HALF = _B // 2
_UNROLL = 16


def _tc_body(pred_ref, targ_ref, bits_ref, mino_ref):
    p = pred_ref[...]
    t = targ_ref[...]
    loss = jnp.maximum(p, 0.0) - p * t + jnp.log1p(jnp.exp(-jnp.abs(p)))

    pos = jnp.sum(t, axis=0, keepdims=True)
    pos_gt = (pos * 2.0 >= float(_B)).astype(jnp.float32)
    majority = t == pos_gt

    mino = jnp.sum(jnp.where(majority, 0.0, loss))
    bits = jnp.where(majority, lax.bitcast_convert_type(loss, jnp.int32),
                     jnp.int32(-1))
    bits_ref[...] = jnp.transpose(bits, (1, 0))

    @pl.when(pl.program_id(0) == 0)
    def _():
        mino_ref[...] = jnp.zeros((1, 1), jnp.float32)

    mino_ref[...] += jnp.reshape(mino, (1, 1))


def _tc_stage(predp, targp):
    return pl.pallas_call(
        _tc_body,
        grid=(_CPAD // _BLK,),
        in_specs=[
            pl.BlockSpec((_B, _BLK), lambda j: (0, j)),
            pl.BlockSpec((_B, _BLK), lambda j: (0, j)),
        ],
        out_specs=[
            pl.BlockSpec((_BLK, _B), lambda j: (j, 0)),
            pl.BlockSpec((1, 1), lambda j: (0, 0)),
        ],
        out_shape=[
            jax.ShapeDtypeStruct((_CPAD, _B), jnp.int32),
            jax.ShapeDtypeStruct((1, 1), jnp.float32),
        ],
        compiler_params=pltpu.CompilerParams(
            dimension_semantics=("arbitrary",),
        ),
    )(predp, targp)


def _sc_body(bits_hbm, out_hbm, stage_ref, slab_ref, res_ref):
    wid = lax.axis_index("s") * _NC + lax.axis_index("c")
    lanes = jnp.arange(_LANES, dtype=jnp.int32)
    zero_i = jnp.zeros((_LANES,), jnp.int32)
    one_i = jnp.ones((_LANES,), jnp.int32)
    zero_f = jnp.zeros((_LANES,), jnp.float32)

    for slab in range(_SLABS_PER_W):
        c0 = (wid * _SLABS_PER_W + slab) * _LANES

        for h in range(2):
            pltpu.sync_copy(
                bits_hbm.at[pl.ds(c0, _LANES), pl.ds(h * _HALF, _HALF)],
                stage_ref)
            for c in range(_LANES):
                col = jnp.full((_LANES,), c, jnp.int32)

                def tr(i, _, c=c, col=col, h=h):
                    v = stage_ref[c, pl.ds(i * _LANES, _LANES)]
                    rows = (h * _HALF + i * _LANES) + lanes
                    plsc.store_scatter(slab_ref, [rows * _LANES + col], v)
                    return 0

                lax.fori_loop(0, _HALF // _LANES, tr, 0, unroll=8)

        def outer(it, carry):
            T, k_vec = carry
            cand = T | (jnp.int32(1) << (30 - it))

            def body0(i, c):
                cnt, kk = c
                for j in range(_UNROLL):
                    v = slab_ref[pl.ds((i * _UNROLL + j) * _LANES, _LANES)]
                    cnt = cnt + jnp.where(v >= cand, one_i, zero_i)
                    kk = kk + jnp.where(v < 0, one_i, zero_i)
                return (cnt, kk)

            def body1(i, c):
                cnt, kk = c
                for j in range(_UNROLL):
                    v = slab_ref[pl.ds((i * _UNROLL + j) * _LANES, _LANES)]
                    cnt = cnt + jnp.where(v >= cand, one_i, zero_i)
                return (cnt, kk)

            cnt, k_vec = lax.cond(
                it == 0,
                lambda: lax.fori_loop(0, _B // _UNROLL, body0,
                                      (zero_i, zero_i), unroll=False),
                lambda: lax.fori_loop(0, _B // _UNROLL, body1,
                                      (zero_i, k_vec), unroll=False),
            )
            return (jnp.where(cnt >= k_vec, cand, T), k_vec)

        T, k_vec = lax.fori_loop(0, 31, outer, (zero_i, zero_i), unroll=False)

        def fin(i, c):
            s, n = c
            for j in range(_UNROLL):
                v = slab_ref[pl.ds((i * _UNROLL + j) * _LANES, _LANES)]
                gt = v > T
                s = s + jnp.where(gt, plsc.bitcast(v, jnp.float32), zero_f)
                n = n + jnp.where(gt, one_i, zero_i)
            return (s, n)

        s, n = lax.fori_loop(0, _B // _UNROLL, fin, (zero_f, zero_i),
                             unroll=False)
        tie = plsc.bitcast(T, jnp.float32)
        extra = (k_vec - n).astype(jnp.float32) * tie
        res_ref[...] = jnp.where(k_vec > 0, s + extra, zero_f)
        pltpu.sync_copy(res_ref, out_hbm.at[pl.ds(c0, _LANES)])


_sc_stage = functools.partial(
    pl.kernel,
    out_type=jax.ShapeDtypeStruct((_CPAD,), jnp.float32),
    mesh=plsc.VectorSubcoreMesh(core_axis_name="c", subcore_axis_name="s"),
    compiler_params=pltpu.CompilerParams(needs_layout_passes=False),
    scratch_types=[
        pltpu.VMEM((_LANES, _HALF), jnp.int32),
        pltpu.VMEM((_B * _LANES,), jnp.int32),
        pltpu.VMEM((_LANES,), jnp.float32),
    ],
)(_sc_body)


def kernel(pred, target):
    pad = _CPAD - _C
    predp = jnp.pad(pred, ((0, 0), (0, pad)))
    targp = jnp.pad(target, ((0, 0), (0, pad)))
    bits, mino = _tc_stage(predp, targp)
    topk = _sc_stage(bits)
    return (mino[0, 0] + jnp.sum(topk)) / jnp.float32(_B * _C)
